# baseline (speedup 1.0000x reference)
import functools

import jax
import jax.numpy as jnp
from jax import lax
from jax.experimental import pallas as pl
from jax.experimental.pallas import tpu as pltpu
from jax.experimental.pallas import tpu_sc as plsc

DIM = 32
GSZ = 640
NBUF = 2
LA = 1


@functools.lru_cache(maxsize=None)
def _build(n_flat, n_pre, n_new, vocab):
    info = plsc.get_sparse_core_info()
    nc, ns, lanes = info.num_cores, info.num_subcores, info.num_lanes
    nw = nc * ns
    assert n_flat % (nw * GSZ) == 0
    per_w = n_flat // nw
    ng = per_w // GSZ
    dump = n_flat

    mesh = plsc.VectorSubcoreMesh(core_axis_name="c", subcore_axis_name="s")

    @functools.partial(
        pl.kernel,
        out_type=jax.ShapeDtypeStruct((n_flat + 8, DIM), jnp.float32),
        mesh=mesh,
        compiler_params=pltpu.CompilerParams(use_tc_tiling_on_sc=False),
        scratch_types=[
            pltpu.VMEM((per_w,), jnp.int32),
            pltpu.VMEM((per_w,), jnp.int32),
            pltpu.VMEM((per_w,), jnp.int32),
            pltpu.VMEM((per_w,), jnp.int32),
            pltpu.VMEM((per_w,), jnp.int32),
            pltpu.VMEM((per_w,), jnp.int32),
            pltpu.VMEM((NBUF, GSZ, DIM), jnp.float32),
            pltpu.VMEM((NBUF, GSZ, DIM), jnp.float32),
            pltpu.SemaphoreType.DMA,
            pltpu.SemaphoreType.DMA((NBUF,)),
            pltpu.SemaphoreType.DMA((NBUF,)),
        ],
    )
    def emb(x_hbm, re_hbm, pre_hbm, new_hbm, out_hbm,
            xv, rv, pidx, nidx, pos_a, pos_b, prebuf, newbuf, sem,
            gsem, ssem):
        c = lax.axis_index("c")
        s = lax.axis_index("s")
        wid = s * nc + c
        base = wid * per_w

        pltpu.sync_copy(x_hbm.at[wid], xv)

        def fire(g, _):
            sl = pl.ds(g * GSZ, GSZ)
            pltpu.async_copy(re_hbm.at[xv.at[sl]], rv.at[sl], sem)
            return 0

        lax.fori_loop(0, ng, fire, 0)

        def drain(g, _):
            sl = pl.ds(g * GSZ, GSZ)
            pltpu.make_async_copy(re_hbm.at[xv.at[sl]], rv.at[sl], sem).wait()
            return 0

        lax.fori_loop(0, ng, drain, 0)

        lane = lax.iota(jnp.int32, lanes)

        def route(i, _):
            sl = pl.ds(i * lanes, lanes)
            r = rv[sl]
            m = r < n_pre
            pidx[sl] = jnp.where(m, r, 0)
            nidx[sl] = jnp.where(m, 0, r - n_pre)
            g = base + i * lanes + lane
            pos_a[sl] = jnp.where(m, g, dump)
            pos_b[sl] = jnp.where(m, dump, g)
            return 0

        lax.fori_loop(0, per_w // lanes, route, 0)

        def start_g(g, b):
            sl = pl.ds(g * GSZ, GSZ)
            pltpu.async_copy(pre_hbm.at[pidx.at[sl]], prebuf.at[b], gsem.at[b])
            pltpu.async_copy(new_hbm.at[nidx.at[sl]], newbuf.at[b], gsem.at[b])

        def wait_g(g, b):
            sl = pl.ds(g * GSZ, GSZ)
            pltpu.make_async_copy(
                pre_hbm.at[pidx.at[sl]], prebuf.at[b], gsem.at[b]).wait()
            pltpu.make_async_copy(
                new_hbm.at[nidx.at[sl]], newbuf.at[b], gsem.at[b]).wait()

        def start_s(g, b):
            sl = pl.ds(g * GSZ, GSZ)
            pltpu.async_copy(prebuf.at[b], out_hbm.at[pos_a.at[sl]], ssem.at[b])
            pltpu.async_copy(newbuf.at[b], out_hbm.at[pos_b.at[sl]], ssem.at[b])

        def wait_s(g, b):
            sl = pl.ds(g * GSZ, GSZ)
            pltpu.make_async_copy(
                prebuf.at[b], out_hbm.at[pos_a.at[sl]], ssem.at[b]).wait()
            pltpu.make_async_copy(
                newbuf.at[b], out_hbm.at[pos_b.at[sl]], ssem.at[b]).wait()

        for pos in range(LA):
            start_g(pos, pos % NBUF)

        def steady(pos, _):
            b_i = lax.rem(pos, NBUF)

            @pl.when(pos >= NBUF)
            def _():
                wait_s(pos - NBUF, b_i)

            start_g(pos, b_i)
            jj = pos - LA
            b_c = lax.rem(jj, NBUF)
            wait_g(jj, b_c)
            start_s(jj, b_c)
            return 0

        lax.fori_loop(LA, ng, steady, 0)

        for jj in range(ng - LA, ng):
            b_c = jj % NBUF
            wait_g(jj, b_c)
            start_s(jj, b_c)
        for g in range(ng - NBUF, ng):
            wait_s(g, g % NBUF)

    return emb


def kernel(x, reordering, pretrained_weight, new_weight):
    b, l = x.shape
    n_flat = b * l
    n_pre = pretrained_weight.shape[0]
    n_new = new_weight.shape[0]
    vocab = reordering.shape[0]
    emb = _build(n_flat, n_pre, n_new, vocab)
    info = plsc.get_sparse_core_info()
    nw = info.num_cores * info.num_subcores
    xf = x.reshape(nw, n_flat // nw)
    out = emb(xf, reordering, pretrained_weight, new_weight)
    return out[:n_flat].reshape(b, l, DIM)


# speedup vs baseline: 1.0007x; 1.0007x over previous
import functools

import jax
import jax.numpy as jnp
from jax import lax
from jax.experimental import pallas as pl
from jax.experimental.pallas import tpu as pltpu
from jax.experimental.pallas import tpu_sc as plsc

DIM = 32
GSZ = 640
NBUF = 2
LA = 1


@functools.lru_cache(maxsize=None)
def _build(n_flat, n_pre, n_new, vocab):
    info = plsc.get_sparse_core_info()
    nc, ns, lanes = info.num_cores, info.num_subcores, info.num_lanes
    nw = nc * ns
    assert n_flat % (nw * GSZ) == 0
    per_w = n_flat // nw
    ng = per_w // GSZ
    dump = n_flat

    mesh = plsc.VectorSubcoreMesh(core_axis_name="c", subcore_axis_name="s")

    @functools.partial(
        pl.kernel,
        out_type=jax.ShapeDtypeStruct((n_flat + 8, DIM), jnp.float32),
        mesh=mesh,
        compiler_params=pltpu.CompilerParams(use_tc_tiling_on_sc=False),
        scratch_types=[
            pltpu.VMEM((per_w,), jnp.int32),
            pltpu.VMEM((per_w,), jnp.int32),
            pltpu.VMEM((per_w,), jnp.int32),
            pltpu.VMEM((per_w,), jnp.int32),
            pltpu.VMEM((per_w,), jnp.int32),
            pltpu.VMEM((per_w,), jnp.int32),
            pltpu.VMEM((NBUF, GSZ, DIM), jnp.float32),
            pltpu.VMEM((NBUF, GSZ, DIM), jnp.float32),
            pltpu.SemaphoreType.DMA,
            pltpu.SemaphoreType.DMA((NBUF,)),
            pltpu.SemaphoreType.DMA((NBUF,)),
        ],
    )
    def emb(x_hbm, re_hbm, pre_hbm, new_hbm, out_hbm,
            xv, rv, pidx, nidx, pos_a, pos_b, prebuf, newbuf, sem,
            gsem, ssem):
        c = lax.axis_index("c")
        s = lax.axis_index("s")
        wid = s * nc + c
        base = wid * per_w

        with jax.named_scope("stage0_xload"):
            pltpu.sync_copy(x_hbm.at[wid], xv)

        def fire(g, _):
            sl = pl.ds(g * GSZ, GSZ)
            pltpu.async_copy(re_hbm.at[xv.at[sl]], rv.at[sl], sem)
            return 0

        def drain(g, _):
            sl = pl.ds(g * GSZ, GSZ)
            pltpu.make_async_copy(re_hbm.at[xv.at[sl]], rv.at[sl], sem).wait()
            return 0

        with jax.named_scope("stage1_remap"):
            lax.fori_loop(0, ng, fire, 0)
            lax.fori_loop(0, ng, drain, 0)

        lane = lax.iota(jnp.int32, lanes)

        def route(i, _):
            sl = pl.ds(i * lanes, lanes)
            r = rv[sl]
            m = r < n_pre
            pidx[sl] = jnp.where(m, r, 0)
            nidx[sl] = jnp.where(m, 0, r - n_pre)
            g = base + i * lanes + lane
            pos_a[sl] = jnp.where(m, g, dump)
            pos_b[sl] = jnp.where(m, dump, g)
            return 0

        with jax.named_scope("stage2_route"):
            lax.fori_loop(0, per_w // lanes, route, 0)

        def start_g(g, b):
            sl = pl.ds(g * GSZ, GSZ)
            pltpu.async_copy(pre_hbm.at[pidx.at[sl]], prebuf.at[b], gsem.at[b])
            pltpu.async_copy(new_hbm.at[nidx.at[sl]], newbuf.at[b], gsem.at[b])

        def wait_g(g, b):
            sl = pl.ds(g * GSZ, GSZ)
            pltpu.make_async_copy(
                pre_hbm.at[pidx.at[sl]], prebuf.at[b], gsem.at[b]).wait()
            pltpu.make_async_copy(
                new_hbm.at[nidx.at[sl]], newbuf.at[b], gsem.at[b]).wait()

        def start_s(g, b):
            sl = pl.ds(g * GSZ, GSZ)
            pltpu.async_copy(prebuf.at[b], out_hbm.at[pos_a.at[sl]], ssem.at[b])
            pltpu.async_copy(newbuf.at[b], out_hbm.at[pos_b.at[sl]], ssem.at[b])

        def wait_s(g, b):
            sl = pl.ds(g * GSZ, GSZ)
            pltpu.make_async_copy(
                prebuf.at[b], out_hbm.at[pos_a.at[sl]], ssem.at[b]).wait()
            pltpu.make_async_copy(
                newbuf.at[b], out_hbm.at[pos_b.at[sl]], ssem.at[b]).wait()

        def steady(pos, _):
            b_i = lax.rem(pos, NBUF)

            @pl.when(pos >= NBUF)
            def _():
                wait_s(pos - NBUF, b_i)

            start_g(pos, b_i)
            jj = pos - LA
            b_c = lax.rem(jj, NBUF)
            wait_g(jj, b_c)
            start_s(jj, b_c)
            return 0

        with jax.named_scope("stage3_rows"):
            for pos in range(LA):
                start_g(pos, pos % NBUF)
            lax.fori_loop(LA, ng, steady, 0)
            for jj in range(ng - LA, ng):
                b_c = jj % NBUF
                wait_g(jj, b_c)
                start_s(jj, b_c)
            for g in range(ng - NBUF, ng):
                wait_s(g, g % NBUF)

    return emb


def kernel(x, reordering, pretrained_weight, new_weight):
    b, l = x.shape
    n_flat = b * l
    n_pre = pretrained_weight.shape[0]
    n_new = new_weight.shape[0]
    vocab = reordering.shape[0]
    emb = _build(n_flat, n_pre, n_new, vocab)
    info = plsc.get_sparse_core_info()
    nw = info.num_cores * info.num_subcores
    xf = x.reshape(nw, n_flat // nw)
    out = emb(xf, reordering, pretrained_weight, new_weight)
    return out[:n_flat].reshape(b, l, DIM)


# speedup vs baseline: 1.0279x; 1.0271x over previous
import functools

import jax
import jax.numpy as jnp
from jax import lax
from jax.experimental import pallas as pl
from jax.experimental.pallas import tpu as pltpu
from jax.experimental.pallas import tpu_sc as plsc

DIM = 32
GSZ = 640
NBUF = 2
LA = 1


@functools.lru_cache(maxsize=None)
def _build(n_flat, n_pre, n_new, vocab):
    info = plsc.get_sparse_core_info()
    nc, ns, lanes = info.num_cores, info.num_subcores, info.num_lanes
    nw = nc * ns
    assert n_flat % (nw * GSZ) == 0
    per_w = n_flat // nw
    ng = per_w // GSZ
    vpg = GSZ // lanes
    dump = n_flat

    mesh = plsc.VectorSubcoreMesh(core_axis_name="c", subcore_axis_name="s")

    @functools.partial(
        pl.kernel,
        out_type=jax.ShapeDtypeStruct((n_flat + 8, DIM), jnp.float32),
        mesh=mesh,
        compiler_params=pltpu.CompilerParams(use_tc_tiling_on_sc=False),
        scratch_types=[
            pltpu.VMEM((per_w,), jnp.int32),
            pltpu.VMEM((per_w,), jnp.int32),
            pltpu.VMEM((NBUF, GSZ, DIM), jnp.float32),
            pltpu.VMEM((NBUF, GSZ, DIM), jnp.float32),
            pltpu.SemaphoreType.DMA,
            pltpu.SemaphoreType.DMA((NBUF,)),
            pltpu.SemaphoreType.DMA((NBUF,)),
        ],
    )
    def emb(x_hbm, re_hbm, pre_hbm, new_hbm, out_hbm,
            xv, rv, prebuf, newbuf, sem, gsem, ssem):
        c = lax.axis_index("c")
        s = lax.axis_index("s")
        wid = s * nc + c
        base = wid * per_w
        lane = lax.iota(jnp.int32, lanes)

        pltpu.sync_copy(x_hbm.at[wid], xv)

        def fire(k, _):
            sl = pl.ds(k * lanes, lanes)
            pltpu.async_copy(re_hbm.at[xv[sl]], rv.at[sl], sem)
            return 0

        lax.fori_loop(0, per_w // lanes, fire, 0)
        pltpu.make_async_copy(re_hbm.at[pl.ds(0, per_w)], rv, sem).wait()

        def start_g(g, b):
            def one(k, _):
                sl = pl.ds(g * GSZ + k * lanes, lanes)
                r = rv[sl]
                m = r < n_pre
                pvec = jnp.where(m, r, 0)
                nvec = jnp.where(m, 0, r - n_pre)
                dsl = pl.ds(k * lanes, lanes)
                pltpu.async_copy(pre_hbm.at[pvec], prebuf.at[b].at[dsl],
                                 gsem.at[b])
                pltpu.async_copy(new_hbm.at[nvec], newbuf.at[b].at[dsl],
                                 gsem.at[b])
                return 0

            lax.fori_loop(0, vpg, one, 0)

        def wait_g(b):
            pltpu.make_async_copy(
                pre_hbm.at[pl.ds(0, GSZ)], prebuf.at[b], gsem.at[b]).wait()
            pltpu.make_async_copy(
                new_hbm.at[pl.ds(0, GSZ)], newbuf.at[b], gsem.at[b]).wait()

        def start_s(g, b):
            def one(k, _):
                sl = pl.ds(g * GSZ + k * lanes, lanes)
                r = rv[sl]
                m = r < n_pre
                gpos = base + g * GSZ + k * lanes + lane
                pos_a = jnp.where(m, gpos, dump)
                pos_b = jnp.where(m, dump, gpos)
                dsl = pl.ds(k * lanes, lanes)
                pltpu.async_copy(prebuf.at[b].at[dsl], out_hbm.at[pos_a],
                                 ssem.at[b])
                pltpu.async_copy(newbuf.at[b].at[dsl], out_hbm.at[pos_b],
                                 ssem.at[b])
                return 0

            lax.fori_loop(0, vpg, one, 0)

        def wait_s(b):
            pltpu.make_async_copy(
                prebuf.at[b], out_hbm.at[pl.ds(0, GSZ)], ssem.at[b]).wait()
            pltpu.make_async_copy(
                newbuf.at[b], out_hbm.at[pl.ds(0, GSZ)], ssem.at[b]).wait()

        for pos in range(LA):
            start_g(pos, pos % NBUF)

        def steady(pos, _):
            b_i = lax.rem(pos, NBUF)

            @pl.when(pos >= NBUF)
            def _():
                wait_s(b_i)

            start_g(pos, b_i)
            jj = pos - LA
            b_c = lax.rem(jj, NBUF)
            wait_g(b_c)
            start_s(jj, b_c)
            return 0

        lax.fori_loop(LA, ng, steady, 0)

        for jj in range(ng - LA, ng):
            b_c = jj % NBUF
            wait_g(b_c)
            start_s(jj, b_c)
        for g in range(ng - NBUF, ng):
            wait_s(g % NBUF)

    return emb


def kernel(x, reordering, pretrained_weight, new_weight):
    b, l = x.shape
    n_flat = b * l
    n_pre = pretrained_weight.shape[0]
    n_new = new_weight.shape[0]
    vocab = reordering.shape[0]
    emb = _build(n_flat, n_pre, n_new, vocab)
    info = plsc.get_sparse_core_info()
    nw = info.num_cores * info.num_subcores
    xf = x.reshape(nw, n_flat // nw)
    out = emb(xf, reordering, pretrained_weight, new_weight)
    return out[:n_flat].reshape(b, l, DIM)


# speedup vs baseline: 1.0627x; 1.0339x over previous
import functools

import jax
import jax.numpy as jnp
from jax import lax
from jax.experimental import pallas as pl
from jax.experimental.pallas import tpu as pltpu
from jax.experimental.pallas import tpu_sc as plsc

DIM = 32
GSZ = 256
NBUF = 2


@functools.lru_cache(maxsize=None)
def _build(n_flat, n_pre, n_new, vocab):
    info = plsc.get_sparse_core_info()
    nc, ns, lanes = info.num_cores, info.num_subcores, info.num_lanes
    nw = nc * ns
    assert n_flat % (nw * GSZ) == 0
    per_w = n_flat // nw
    ng = per_w // GSZ
    vpg = GSZ // lanes
    dump = n_flat

    mesh = plsc.VectorSubcoreMesh(core_axis_name="c", subcore_axis_name="s")

    @functools.partial(
        pl.kernel,
        out_type=jax.ShapeDtypeStruct((n_flat + 8, DIM), jnp.float32),
        mesh=mesh,
        compiler_params=pltpu.CompilerParams(use_tc_tiling_on_sc=False),
        scratch_types=[
            pltpu.VMEM((per_w,), jnp.int32),
            pltpu.VMEM((per_w,), jnp.int32),
            pltpu.VMEM((NBUF, GSZ, DIM), jnp.float32),
            pltpu.VMEM((NBUF, GSZ, DIM), jnp.float32),
            pltpu.VMEM_SHARED((vocab,), jnp.int32),
            pltpu.SemaphoreType.DMA,
            pltpu.SemaphoreType.DMA((NBUF,)),
            pltpu.SemaphoreType.DMA((NBUF,)),
            pltpu.SemaphoreType.DMA((NBUF,)),
        ],
    )
    def emb(x_hbm, re_hbm, pre_hbm, new_hbm, out_hbm,
            xv, rv, prebuf, newbuf, re_sp, sem, gsem, asem, bsem):
        c = lax.axis_index("c")
        s = lax.axis_index("s")
        wid = s * nc + c
        base = wid * per_w
        lane = lax.iota(jnp.int32, lanes)

        @pl.when(s == 0)
        def _():
            pltpu.sync_copy(re_hbm, re_sp)

        pltpu.sync_copy(x_hbm.at[wid], xv)
        plsc.subcore_barrier()

        def fire(k, _):
            sl = pl.ds(k * lanes, lanes)
            pltpu.async_copy(re_sp.at[xv[sl]], rv.at[sl], sem)
            return 0

        lax.fori_loop(0, per_w // lanes, fire, 0)
        pltpu.make_async_copy(re_sp.at[pl.ds(0, per_w)], rv, sem).wait()

        def start_g(g, b):
            def one(k, _):
                sl = pl.ds(g * GSZ + k * lanes, lanes)
                r = rv[sl]
                m = r < n_pre
                pvec = jnp.where(m, r, 0)
                nvec = jnp.where(m, 0, r - n_pre)
                dsl = pl.ds(k * lanes, lanes)
                pltpu.async_copy(pre_hbm.at[pvec], prebuf.at[b].at[dsl],
                                 gsem.at[b])
                pltpu.async_copy(new_hbm.at[nvec], newbuf.at[b].at[dsl],
                                 gsem.at[b])
                return 0

            lax.fori_loop(0, vpg, one, 0)

        def wait_g(b):
            pltpu.make_async_copy(
                pre_hbm.at[pl.ds(0, GSZ)], prebuf.at[b], gsem.at[b]).wait()
            pltpu.make_async_copy(
                new_hbm.at[pl.ds(0, GSZ)], newbuf.at[b], gsem.at[b]).wait()

        def start_a(g, b):
            def one(k, _):
                sl = pl.ds(g * GSZ + k * lanes, lanes)
                r = rv[sl]
                gpos = base + g * GSZ + k * lanes + lane
                pos = jnp.where(r < n_pre, gpos, dump)
                dsl = pl.ds(k * lanes, lanes)
                pltpu.async_copy(prebuf.at[b].at[dsl], out_hbm.at[pos],
                                 asem.at[b])
                return 0

            lax.fori_loop(0, vpg, one, 0)

        def wait_a(b):
            pltpu.make_async_copy(
                prebuf.at[b], out_hbm.at[pl.ds(0, GSZ)], asem.at[b]).wait()

        def start_b(g, b):
            def one(k, _):
                sl = pl.ds(g * GSZ + k * lanes, lanes)
                r = rv[sl]
                gpos = base + g * GSZ + k * lanes + lane
                pos = jnp.where(r < n_pre, dump, gpos)
                dsl = pl.ds(k * lanes, lanes)
                pltpu.async_copy(newbuf.at[b].at[dsl], out_hbm.at[pos],
                                 bsem.at[b])
                return 0

            lax.fori_loop(0, vpg, one, 0)

        def wait_b(b):
            pltpu.make_async_copy(
                newbuf.at[b], out_hbm.at[pl.ds(0, GSZ)], bsem.at[b]).wait()

        start_g(0, 0)

        def steady(g, _):
            b = lax.rem(g, NBUF)
            b_n = lax.rem(g + 1, NBUF)

            @pl.when(g >= 1)
            def _():
                wait_a(b_n)
                wait_b(b_n)

            @pl.when(g + 1 < ng)
            def _():
                start_g(g + 1, b_n)

            wait_g(b)
            start_a(g, b)
            start_b(g, b)
            return 0

        lax.fori_loop(0, ng, steady, 0)
        wait_a(lax.rem(ng - 1, NBUF))
        wait_b(lax.rem(ng - 1, NBUF))

    return emb


def kernel(x, reordering, pretrained_weight, new_weight):
    b, l = x.shape
    n_flat = b * l
    n_pre = pretrained_weight.shape[0]
    n_new = new_weight.shape[0]
    vocab = reordering.shape[0]
    emb = _build(n_flat, n_pre, n_new, vocab)
    info = plsc.get_sparse_core_info()
    nw = info.num_cores * info.num_subcores
    xf = x.reshape(nw, n_flat // nw)
    out = emb(xf, reordering, pretrained_weight, new_weight)
    return out[:n_flat].reshape(b, l, DIM)
